# baseline (speedup 1.0000x reference)
import functools

import jax
import jax.numpy as jnp
from jax import lax
from jax.experimental import pallas as pl
from jax.experimental.pallas import tpu as pltpu
from jax.experimental.pallas import tpu_sc as plsc

N = 10000
NP = 10240
D = 128
G = 16
E = 320000
NW = 32
CH = 64
CPW = 160
EP = NW * CPW * CH
PAD = EP - E
NROWS = EP // CH
SPT = NP // 16
GPC = 40
NGRP = CPW // GPC

_mesh = plsc.VectorSubcoreMesh(core_axis_name="c", subcore_axis_name="s",
                               num_cores=2, num_subcores=16)
_sc_params = pltpu.CompilerParams(needs_layout_passes=False)


def _deg_body(col_hbm, ew_hbm, degb_hbm, col_v, ew_v, stage_v, bbuf_v, dsem,
              deg_sh):
    cid = lax.axis_index("c")
    sid = lax.axis_index("s")
    wid = sid * 2 + cid
    pltpu.sync_copy(col_hbm.at[pl.ds(wid * CPW, CPW)], col_v)
    pltpu.sync_copy(ew_hbm.at[pl.ds(wid * CPW, CPW)], ew_v)

    zero16 = jnp.zeros((16,), jnp.float32)
    for k in range(SPT // 16):
        stage_v[pl.ds(k * 16, 16)] = zero16
    pltpu.sync_copy(stage_v, deg_sh.at[pl.ds(sid * SPT, SPT)])
    plsc.subcore_barrier()

    def sc_issue(j, c):
        pltpu.async_copy(ew_v.at[j], deg_sh.at[col_v.at[j]], dsem, add=True)
        return c
    lax.fori_loop(0, CPW, sc_issue, 0)

    def sc_drain(j, c):
        pltpu.make_async_copy(ew_v.at[0], deg_sh.at[col_v.at[0]], dsem).wait()
        return c
    lax.fori_loop(0, CPW, sc_drain, 0)
    plsc.subcore_barrier()

    start = sid * SPT
    pltpu.sync_copy(deg_sh.at[pl.ds(start, SPT)], stage_v)
    for g in range(SPT // 128):
        def fill(i, c):
            s = plsc.load_gather(
                stage_v, [jnp.zeros((16,), jnp.int32) + (g * 128 + i)])
            for k in range(8):
                bbuf_v[i, pl.ds(k * 16, 16)] = s
            return c
        lax.fori_loop(0, 128, fill, 0)
        pltpu.sync_copy(bbuf_v, degb_hbm.at[cid, pl.ds(start + g * 128, 128)])


_deg_call = functools.partial(
    pl.kernel,
    out_type=jax.ShapeDtypeStruct((2, NP, D), jnp.float32),
    mesh=_mesh,
    compiler_params=_sc_params,
    scratch_types=[
        pltpu.VMEM((CPW, CH), jnp.int32),
        pltpu.VMEM((CPW, CH), jnp.float32),
        pltpu.VMEM((SPT,), jnp.float32),
        pltpu.VMEM((128, D), jnp.float32),
        pltpu.SemaphoreType.DMA,
        pltpu.VMEM_SHARED((NP,), jnp.float32),
    ],
)(_deg_body)


def _xs_body(node_ref, w_ref, degp_ref, xs_ref, dis_ref):
    deg = (degp_ref[0, pl.ds(0, N), :] + degp_ref[1, pl.ds(0, N), :] + 1.0)
    dis = jnp.where(deg > 0, lax.rsqrt(jnp.maximum(deg, 1e-12)), 0.0)
    x = jnp.dot(node_ref[...], w_ref[...], preferred_element_type=jnp.float32)
    dis_ref[...] = dis
    xs_ref[pl.ds(0, N), :] = x * dis
    xs_ref[pl.ds(N, NP - N), :] = jnp.zeros((NP - N, D), jnp.float32)


_xs_call = pl.pallas_call(
    _xs_body,
    out_shape=[jax.ShapeDtypeStruct((NP, D), jnp.float32),
               jax.ShapeDtypeStruct((N, D), jnp.float32)],
)


def _msg_body(xs_hbm, row_hbm, col_hbm, ew_hbm, accp_hbm,
              ridx_v, cidx_v, ew_v, rb0, rb1, rb2, rb3,
              g0, g1, g2, g3, s0, s1, s2, s3, acc_sh):
    cid = lax.axis_index("c")
    sid = lax.axis_index("s")
    wid = sid * 2 + cid

    rbufs = (rb0, rb1, rb2, rb3)
    gsems = (g0, g1, g2, g3)
    ssems = (s0, s1, s2, s3)

    zero16 = jnp.zeros((16,), jnp.float32)

    def zrow(r, c):
        for k in range(8):
            rb0[r, pl.ds(k * 16, 16)] = zero16
        return c
    lax.fori_loop(0, CH, zrow, 0)
    start = sid * SPT
    for t in range(SPT // CH):
        pltpu.sync_copy(rb0, acc_sh.at[pl.ds(start + t * CH, CH)])
    plsc.subcore_barrier()

    def gather_start(l, buf, sem):
        pltpu.async_copy(xs_hbm.at[ridx_v.at[l]], buf, sem)

    def gather_wait(l, buf, sem):
        pltpu.make_async_copy(xs_hbm.at[ridx_v.at[l]], buf, sem).wait()

    def scatter_start(l, buf, sem):
        pltpu.async_copy(buf, acc_sh.at[cidx_v.at[l]], sem, add=True)

    def scatter_wait(l, buf, sem):
        pltpu.make_async_copy(buf, acc_sh.at[cidx_v.at[l]], sem).wait()

    for grp in range(NGRP):
        base = wid * CPW + grp * GPC
        pltpu.sync_copy(row_hbm.at[pl.ds(base, GPC)], ridx_v)
        pltpu.sync_copy(col_hbm.at[pl.ds(base, GPC)], cidx_v)
        pltpu.sync_copy(ew_hbm.at[pl.ds(base, GPC)], ew_v)
        gather_start(0, rb0, g0)
        gather_start(1, rb1, g1)

        def quad(t, c):
            for i in range(4):
                l = 4 * t + i
                lv = jnp.zeros((16,), jnp.int32) + l
                gather_wait(l, rbufs[i], gsems[i])

                @plsc.parallel_loop(0, CH // 16, unroll=2)
                def eblk(q):
                    sv = plsc.load_gather(
                        ew_v, [lv, q * 16 + lax.iota(jnp.int32, 16)])
                    for u in range(16):
                        s = lax.gather(
                            sv, jnp.full((16, 1), u, jnp.int32),
                            lax.GatherDimensionNumbers(
                                offset_dims=(), collapsed_slice_dims=(0,),
                                start_index_map=(0,)),
                            (1,), mode=lax.GatherScatterMode.PROMISE_IN_BOUNDS)
                        e = q * 16 + u
                        for k in range(8):
                            rbufs[i][e, pl.ds(k * 16, 16)] = (
                                rbufs[i][e, pl.ds(k * 16, 16)] * s)

                scatter_start(l, rbufs[i], ssems[i])
                ni = (i + 2) % 4

                @pl.when(l + 2 < GPC)
                def _():
                    @pl.when(l >= 2)
                    def _():
                        scatter_wait(l - 2, rbufs[ni], ssems[ni])
                    gather_start(l + 2, rbufs[ni], gsems[ni])
            return c
        lax.fori_loop(0, GPC // 4, quad, 0)
        scatter_wait(GPC - 4, rb0, s0)
        scatter_wait(GPC - 3, rb1, s1)
        scatter_wait(GPC - 2, rb2, s2)
        scatter_wait(GPC - 1, rb3, s3)
    plsc.subcore_barrier()
    pltpu.sync_copy(acc_sh.at[pl.ds(start, SPT)],
                    accp_hbm.at[cid, pl.ds(start, SPT)])


_msg_call = functools.partial(
    pl.kernel,
    out_type=jax.ShapeDtypeStruct((2, NP, D), jnp.float32),
    mesh=_mesh,
    compiler_params=_sc_params,
    scratch_types=[
        pltpu.VMEM((GPC, CH), jnp.int32),
        pltpu.VMEM((GPC, CH), jnp.int32),
        pltpu.VMEM((GPC, CH), jnp.float32),
        pltpu.VMEM((CH, D), jnp.float32),
        pltpu.VMEM((CH, D), jnp.float32),
        pltpu.VMEM((CH, D), jnp.float32),
        pltpu.VMEM((CH, D), jnp.float32),
        pltpu.SemaphoreType.DMA,
        pltpu.SemaphoreType.DMA,
        pltpu.SemaphoreType.DMA,
        pltpu.SemaphoreType.DMA,
        pltpu.SemaphoreType.DMA,
        pltpu.SemaphoreType.DMA,
        pltpu.SemaphoreType.DMA,
        pltpu.SemaphoreType.DMA,
        pltpu.VMEM_SHARED((NP, D), jnp.float32),
    ],
)(_msg_body)


def _gn_body(accp_ref, xs_ref, dis_ref, oh_ref, b_ref, gnw_ref, gnb_ref,
             gms_ref, out_ref):
    dis = dis_ref[...]
    pre = dis * (accp_ref[0, pl.ds(0, N), :] + accp_ref[1, pl.ds(0, N), :]
                 + xs_ref[pl.ds(0, N), :]) + b_ref[...]
    onehot = oh_ref[...]
    cnt = jnp.maximum(
        lax.dot_general(onehot, jnp.ones((N, 1), jnp.float32),
                        (((0,), (0,)), ((), ())),
                        preferred_element_type=jnp.float32), 1.0)
    sums = lax.dot_general(onehot, pre, (((0,), (0,)), ((), ())),
                           preferred_element_type=jnp.float32)
    mean = sums / cnt
    meanb = jnp.dot(onehot, mean, preferred_element_type=jnp.float32)
    centered = pre - meanb * gms_ref[...]
    var = lax.dot_general(onehot, centered * centered,
                          (((0,), (0,)), ((), ())),
                          preferred_element_type=jnp.float32) / cnt
    varb = jnp.dot(onehot, var, preferred_element_type=jnp.float32)
    out_ref[...] = jnp.maximum(
        gnw_ref[...] * centered * lax.rsqrt(varb + 1e-5) + gnb_ref[...], 0.0)


_gn_call = pl.pallas_call(
    _gn_body,
    out_shape=jax.ShapeDtypeStruct((N, D), jnp.float32),
)


def kernel(node, edge_index, edge_attr, batch_ptr, W, b, gn_weight, gn_bias,
           gn_mean_scale):
    row = edge_index[0].astype(jnp.int32)
    col = edge_index[1].astype(jnp.int32)
    pidx = jnp.arange(PAD, dtype=jnp.int32)
    rowp = jnp.concatenate([row, pidx]).reshape(NROWS, CH)
    colp = jnp.concatenate([col, pidx]).reshape(NROWS, CH)
    ewp = jnp.concatenate(
        [edge_attr.astype(jnp.float32), jnp.zeros((PAD,), jnp.float32)]
    ).reshape(NROWS, CH)

    degp = _deg_call(colp, ewp)
    xs, dis = _xs_call(node, W, degp)
    accp = _msg_call(xs, rowp, colp, ewp)
    onehot = (batch_ptr.astype(jnp.int32)[:, None]
              == jnp.arange(G, dtype=jnp.int32)[None, :]).astype(jnp.float32)
    out = _gn_call(accp, xs, dis, onehot,
                   b.reshape(1, D), gn_weight.reshape(1, D),
                   gn_bias.reshape(1, D), gn_mean_scale.reshape(1, D))
    return out


# speedup vs baseline: 1.0120x; 1.0120x over previous
import functools

import jax
import jax.numpy as jnp
from jax import lax
from jax.experimental import pallas as pl
from jax.experimental.pallas import tpu as pltpu
from jax.experimental.pallas import tpu_sc as plsc

N = 10000
NP = 10240
D = 128
G = 16
E = 320000
NW = 32
CH = 64
CPW = 160
EP = NW * CPW * CH
PAD = EP - E
NROWS = EP // CH
SPT = NP // 16
GPC = 40
NGRP = CPW // GPC

_mesh = plsc.VectorSubcoreMesh(core_axis_name="c", subcore_axis_name="s",
                               num_cores=2, num_subcores=16)
_sc_params = pltpu.CompilerParams(needs_layout_passes=False)


def _deg_body(col_hbm, ew_hbm, degb_hbm, col_v, ew_v, stage_v, bbuf_v, dsem,
              deg_sh):
    cid = lax.axis_index("c")
    sid = lax.axis_index("s")
    wid = sid * 2 + cid
    pltpu.sync_copy(col_hbm.at[pl.ds(wid * CPW, CPW)], col_v)
    pltpu.sync_copy(ew_hbm.at[pl.ds(wid * CPW, CPW)], ew_v)

    zero16 = jnp.zeros((16,), jnp.float32)
    for k in range(SPT // 16):
        stage_v[pl.ds(k * 16, 16)] = zero16
    pltpu.sync_copy(stage_v, deg_sh.at[pl.ds(sid * SPT, SPT)])
    plsc.subcore_barrier()

    def sc_issue(j, c):
        pltpu.async_copy(ew_v.at[j], deg_sh.at[col_v.at[j]], dsem, add=True)
        return c
    lax.fori_loop(0, CPW, sc_issue, 0)

    def sc_drain(j, c):
        pltpu.make_async_copy(ew_v.at[0], deg_sh.at[col_v.at[0]], dsem).wait()
        return c
    lax.fori_loop(0, CPW, sc_drain, 0)
    plsc.subcore_barrier()

    start = sid * SPT
    pltpu.sync_copy(deg_sh.at[pl.ds(start, SPT)], stage_v)
    for g in range(SPT // 128):
        def fill(i, c):
            s = plsc.load_gather(
                stage_v, [jnp.zeros((16,), jnp.int32) + (g * 128 + i)])
            sb = plsc.pack(s, s, format=plsc.PackFormat.INTERLEAVED)
            for k in range(4):
                bbuf_v[i, pl.ds(k * 32, 32)] = sb
            return c
        lax.fori_loop(0, 128, fill, 0)
        pltpu.sync_copy(bbuf_v, degb_hbm.at[cid, pl.ds(start + g * 128, 128)])


_deg_call = functools.partial(
    pl.kernel,
    out_type=jax.ShapeDtypeStruct((2, NP, D), jnp.bfloat16),
    mesh=_mesh,
    compiler_params=_sc_params,
    scratch_types=[
        pltpu.VMEM((CPW, CH), jnp.int32),
        pltpu.VMEM((CPW, CH), jnp.float32),
        pltpu.VMEM((SPT,), jnp.float32),
        pltpu.VMEM((128, D), jnp.bfloat16),
        pltpu.SemaphoreType.DMA,
        pltpu.VMEM_SHARED((NP,), jnp.float32),
    ],
)(_deg_body)


def _dis_of(degp_ref):
    deg = (degp_ref[0, pl.ds(0, N), :].astype(jnp.float32)
           + degp_ref[1, pl.ds(0, N), :].astype(jnp.float32) + 1.0)
    return jnp.where(deg > 0, lax.rsqrt(jnp.maximum(deg, 1e-12)), 0.0)


def _xs_body(node_ref, w_ref, degp_ref, xs_ref):
    x = jnp.dot(node_ref[...], w_ref[...], preferred_element_type=jnp.float32)
    xs_ref[pl.ds(0, N), :] = x * _dis_of(degp_ref)
    xs_ref[pl.ds(N, NP - N), :] = jnp.zeros((NP - N, D), jnp.float32)


_xs_call = pl.pallas_call(
    _xs_body,
    out_shape=jax.ShapeDtypeStruct((NP, D), jnp.float32),
)


def _msg_body(xs_hbm, row_hbm, col_hbm, ew_hbm, accp_hbm,
              ridx_v, cidx_v, ew_v, rb0, rb1, rb2, rb3,
              g0, g1, g2, g3, s0, s1, s2, s3, acc_sh):
    cid = lax.axis_index("c")
    sid = lax.axis_index("s")
    wid = sid * 2 + cid

    rbufs = (rb0, rb1, rb2, rb3)
    gsems = (g0, g1, g2, g3)
    ssems = (s0, s1, s2, s3)

    zero16 = jnp.zeros((16,), jnp.float32)

    def zrow(r, c):
        for k in range(8):
            rb0[r, pl.ds(k * 16, 16)] = zero16
        return c
    lax.fori_loop(0, CH, zrow, 0)
    start = sid * SPT
    for t in range(SPT // CH):
        pltpu.sync_copy(rb0, acc_sh.at[pl.ds(start + t * CH, CH)])
    plsc.subcore_barrier()

    def gather_start(l, buf, sem):
        pltpu.async_copy(xs_hbm.at[ridx_v.at[l]], buf, sem)

    def gather_wait(l, buf, sem):
        pltpu.make_async_copy(xs_hbm.at[ridx_v.at[l]], buf, sem).wait()

    def scatter_start(l, buf, sem):
        pltpu.async_copy(buf, acc_sh.at[cidx_v.at[l]], sem, add=True)

    def scatter_wait(l, buf, sem):
        pltpu.make_async_copy(buf, acc_sh.at[cidx_v.at[l]], sem).wait()

    for grp in range(NGRP):
        base = wid * CPW + grp * GPC
        pltpu.sync_copy(row_hbm.at[pl.ds(base, GPC)], ridx_v)
        pltpu.sync_copy(col_hbm.at[pl.ds(base, GPC)], cidx_v)
        pltpu.sync_copy(ew_hbm.at[pl.ds(base, GPC)], ew_v)
        gather_start(0, rb0, g0)
        gather_start(1, rb1, g1)

        def quad(t, c):
            for i in range(4):
                l = 4 * t + i
                lv = jnp.zeros((16,), jnp.int32) + l
                gather_wait(l, rbufs[i], gsems[i])

                @plsc.parallel_loop(0, CH // 16, unroll=2)
                def eblk(q):
                    sv = plsc.load_gather(
                        ew_v, [lv, q * 16 + lax.iota(jnp.int32, 16)])
                    for u in range(16):
                        s = lax.gather(
                            sv, jnp.full((16, 1), u, jnp.int32),
                            lax.GatherDimensionNumbers(
                                offset_dims=(), collapsed_slice_dims=(0,),
                                start_index_map=(0,)),
                            (1,), mode=lax.GatherScatterMode.PROMISE_IN_BOUNDS)
                        e = q * 16 + u
                        for k in range(8):
                            rbufs[i][e, pl.ds(k * 16, 16)] = (
                                rbufs[i][e, pl.ds(k * 16, 16)] * s)

                scatter_start(l, rbufs[i], ssems[i])
                ni = (i + 2) % 4

                @pl.when(l + 2 < GPC)
                def _():
                    @pl.when(l >= 2)
                    def _():
                        scatter_wait(l - 2, rbufs[ni], ssems[ni])
                    gather_start(l + 2, rbufs[ni], gsems[ni])
            return c
        lax.fori_loop(0, GPC // 4, quad, 0)
        scatter_wait(GPC - 4, rb0, s0)
        scatter_wait(GPC - 3, rb1, s1)
        scatter_wait(GPC - 2, rb2, s2)
        scatter_wait(GPC - 1, rb3, s3)
    plsc.subcore_barrier()
    pltpu.sync_copy(acc_sh.at[pl.ds(start, SPT)],
                    accp_hbm.at[cid, pl.ds(start, SPT)])


_msg_call = functools.partial(
    pl.kernel,
    out_type=jax.ShapeDtypeStruct((2, NP, D), jnp.float32),
    mesh=_mesh,
    compiler_params=_sc_params,
    scratch_types=[
        pltpu.VMEM((GPC, CH), jnp.int32),
        pltpu.VMEM((GPC, CH), jnp.int32),
        pltpu.VMEM((GPC, CH), jnp.float32),
        pltpu.VMEM((CH, D), jnp.float32),
        pltpu.VMEM((CH, D), jnp.float32),
        pltpu.VMEM((CH, D), jnp.float32),
        pltpu.VMEM((CH, D), jnp.float32),
        pltpu.SemaphoreType.DMA,
        pltpu.SemaphoreType.DMA,
        pltpu.SemaphoreType.DMA,
        pltpu.SemaphoreType.DMA,
        pltpu.SemaphoreType.DMA,
        pltpu.SemaphoreType.DMA,
        pltpu.SemaphoreType.DMA,
        pltpu.SemaphoreType.DMA,
        pltpu.VMEM_SHARED((NP, D), jnp.float32),
    ],
)(_msg_body)


def _gn_body(accp_ref, xs_ref, degp_ref, oh_ref, b_ref, gnw_ref, gnb_ref,
             gms_ref, out_ref):
    dis = _dis_of(degp_ref)
    pre = dis * (accp_ref[0, pl.ds(0, N), :] + accp_ref[1, pl.ds(0, N), :]
                 + xs_ref[pl.ds(0, N), :]) + b_ref[...]
    onehot = oh_ref[...]
    cnt = jnp.maximum(
        lax.dot_general(onehot, jnp.ones((N, 1), jnp.float32),
                        (((0,), (0,)), ((), ())),
                        preferred_element_type=jnp.float32), 1.0)
    sums = lax.dot_general(onehot, pre, (((0,), (0,)), ((), ())),
                           preferred_element_type=jnp.float32)
    mean = sums / cnt
    meanb = jnp.dot(onehot, mean, preferred_element_type=jnp.float32)
    centered = pre - meanb * gms_ref[...]
    var = lax.dot_general(onehot, centered * centered,
                          (((0,), (0,)), ((), ())),
                          preferred_element_type=jnp.float32) / cnt
    varb = jnp.dot(onehot, var, preferred_element_type=jnp.float32)
    out_ref[...] = jnp.maximum(
        gnw_ref[...] * centered * lax.rsqrt(varb + 1e-5) + gnb_ref[...], 0.0)


_gn_call = pl.pallas_call(
    _gn_body,
    out_shape=jax.ShapeDtypeStruct((N, D), jnp.float32),
)


def kernel(node, edge_index, edge_attr, batch_ptr, W, b, gn_weight, gn_bias,
           gn_mean_scale):
    row = edge_index[0].astype(jnp.int32)
    col = edge_index[1].astype(jnp.int32)
    pidx = jnp.arange(PAD, dtype=jnp.int32)
    rowp = jnp.concatenate([row, pidx]).reshape(NROWS, CH)
    colp = jnp.concatenate([col, pidx]).reshape(NROWS, CH)
    ewp = jnp.concatenate(
        [edge_attr.astype(jnp.float32), jnp.zeros((PAD,), jnp.float32)]
    ).reshape(NROWS, CH)

    degp = _deg_call(colp, ewp)
    xs = _xs_call(node, W, degp)
    accp = _msg_call(xs, rowp, colp, ewp)
    onehot = (batch_ptr.astype(jnp.int32)[:, None]
              == jnp.arange(G, dtype=jnp.int32)[None, :]).astype(jnp.float32)
    out = _gn_call(accp, xs, degp, onehot,
                   b.reshape(1, D), gn_weight.reshape(1, D),
                   gn_bias.reshape(1, D), gn_mean_scale.reshape(1, D))
    return out
